# Initial kernel scaffold; baseline (speedup 1.0000x reference)
import jax
import jax.numpy as jnp
from jax.experimental import pallas as pl


def kernel(x):
    raise NotImplementedError("write your pallas kernel here")


# speedup vs baseline: 28.5448x; 28.5448x over previous
import functools

import jax
import jax.numpy as jnp
from jax.experimental import pallas as pl
from jax.experimental.pallas import tpu as pltpu

B, N, C, K = 2, 4096, 16, 16
R = 256


def _adj_kernel(xr_ref, xa_ref, out_ref):
    xr = xr_ref[0]
    xa = xa_ref[0]
    sq_r = jnp.sum(xr * xr, axis=-1, keepdims=True)
    sq_a = jnp.sum(xa * xa, axis=-1, keepdims=True).T
    dots = jax.lax.dot_general(
        xr, xa, (((1,), (1,)), ((), ())),
        preferred_element_type=jnp.float32)
    d2 = jnp.maximum(sq_r + sq_a - 2.0 * dots, 0.0)

    big = jnp.float32(jnp.inf)
    t = jnp.full((R, 1), -1.0, dtype=jnp.float32)
    for _ in range(K):
        m = jnp.where(d2 > t, d2, big)
        t = jnp.min(m, axis=-1, keepdims=True)

    out_ref[0] = (d2 <= t).astype(jnp.float32)


@jax.jit
def kernel(x):
    grid = (B, N // R)
    return pl.pallas_call(
        _adj_kernel,
        grid=grid,
        in_specs=[
            pl.BlockSpec((1, R, C), lambda b, i: (b, i, 0)),
            pl.BlockSpec((1, N, C), lambda b, i: (b, 0, 0)),
        ],
        out_specs=pl.BlockSpec((1, R, N), lambda b, i: (b, i, 0)),
        out_shape=jax.ShapeDtypeStruct((B, N, N), jnp.float32),
        compiler_params=pltpu.CompilerParams(
            dimension_semantics=("parallel", "arbitrary"),
        ),
    )(x, x)


# speedup vs baseline: 47.2761x; 1.6562x over previous
import jax
import jax.numpy as jnp
from jax.experimental import pallas as pl
from jax.experimental.pallas import tpu as pltpu

B, N, C, K = 2, 4096, 16, 16
R = 256
G = 4
W = N // G


def _adj_kernel(xr_ref, xa_ref, out_ref):
    xr = xr_ref[0]
    xa = xa_ref[0]
    sq_r = jnp.sum(xr * xr, axis=-1, keepdims=True)
    sq_a = jnp.sum(xa * xa, axis=-1, keepdims=True).T
    dots = jax.lax.dot_general(
        xr, xa, (((1,), (1,)), ((), ())),
        preferred_element_type=jnp.float32)
    d2 = jnp.maximum(sq_r + sq_a - 2.0 * dots, 0.0)

    gmin = d2[:, :W]
    for g in range(1, G):
        gmin = jnp.minimum(gmin, d2[:, g * W:(g + 1) * W])

    big = jnp.float32(jnp.inf)
    t = jnp.full((R, 1), -1.0, dtype=jnp.float32)
    for _ in range(K):
        m = jnp.where(gmin > t, gmin, big)
        t = jnp.min(m, axis=-1, keepdims=True)

    cnt = jnp.sum((d2 <= t).astype(jnp.float32), axis=-1, keepdims=True)
    over = cnt - jnp.float32(K)

    def cond(carry):
        t_c, over_c = carry
        return jnp.max(over_c) > 0.0

    def body(carry):
        t_c, over_c = carry
        tm = jnp.max(jnp.where(d2 < t_c, d2, -1.0), axis=-1, keepdims=True)
        need = over_c > 0.0
        return (jnp.where(need, tm, t_c),
                jnp.where(need, over_c - 1.0, over_c))

    t, _ = jax.lax.while_loop(cond, body, (t, over))

    out_ref[0] = (d2 <= t).astype(jnp.float32)


@jax.jit
def kernel(x):
    grid = (B, N // R)
    return pl.pallas_call(
        _adj_kernel,
        grid=grid,
        in_specs=[
            pl.BlockSpec((1, R, C), lambda b, i: (b, i, 0)),
            pl.BlockSpec((1, N, C), lambda b, i: (b, 0, 0)),
        ],
        out_specs=pl.BlockSpec((1, R, N), lambda b, i: (b, i, 0)),
        out_shape=jax.ShapeDtypeStruct((B, N, N), jnp.float32),
        compiler_params=pltpu.CompilerParams(
            dimension_semantics=("parallel", "arbitrary"),
        ),
    )(x, x)


# speedup vs baseline: 49.5537x; 1.0482x over previous
import jax
import jax.numpy as jnp
from jax.experimental import pallas as pl
from jax.experimental.pallas import tpu as pltpu

B, N, C, K = 2, 4096, 16, 16
R = 256
G = 8
W = N // G
NEG = -1e30


def _adj_kernel(xr_ref, xa_ref, out_ref):
    xr = xr_ref[0]
    xa = xa_ref[0]
    sq_a_half = 0.5 * jnp.sum(xa * xa, axis=-1, keepdims=True).T
    dots = jax.lax.dot_general(
        xr, xa, (((1,), (1,)), ((), ())),
        preferred_element_type=jnp.float32)
    s = sq_a_half - dots

    gmin = s[:, :W]
    for g in range(1, G):
        gmin = jnp.minimum(gmin, s[:, g * W:(g + 1) * W])

    big = jnp.float32(jnp.inf)
    t = jnp.full((R, 1), -jnp.inf, dtype=jnp.float32)
    for _ in range(K):
        m = jnp.where(gmin > t, gmin, big)
        t = jnp.min(m, axis=-1, keepdims=True)

    cnt = jnp.sum((s <= t).astype(jnp.float32), axis=-1, keepdims=True)
    over = cnt - jnp.float32(K)

    def cond(carry):
        _, over_c = carry
        return jnp.max(over_c) > 0.0

    def body(carry):
        t_c, over_c = carry
        tm = jnp.max(jnp.where(s < t_c, s, NEG), axis=-1, keepdims=True)
        need = over_c > 0.0
        return (jnp.where(need, tm, t_c),
                jnp.where(need, over_c - 1.0, over_c))

    t, _ = jax.lax.while_loop(cond, body, (t, over))

    out_ref[0] = (s <= t).astype(jnp.float32)


@jax.jit
def kernel(x):
    grid = (B, N // R)
    return pl.pallas_call(
        _adj_kernel,
        grid=grid,
        in_specs=[
            pl.BlockSpec((1, R, C), lambda b, i: (b, i, 0)),
            pl.BlockSpec((1, N, C), lambda b, i: (b, 0, 0)),
        ],
        out_specs=pl.BlockSpec((1, R, N), lambda b, i: (b, i, 0)),
        out_shape=jax.ShapeDtypeStruct((B, N, N), jnp.float32),
        compiler_params=pltpu.CompilerParams(
            dimension_semantics=("parallel", "arbitrary"),
        ),
    )(x, x)


# speedup vs baseline: 61.2685x; 1.2364x over previous
import jax
import jax.numpy as jnp
from jax.experimental import pallas as pl
from jax.experimental.pallas import tpu as pltpu

B, N, C, K = 2, 4096, 16, 16
R = 1024
G = 8
W = N // G
NEG = -1e30


def _adj_kernel(xr_ref, xa_ref, out_ref):
    xr = xr_ref[0]
    xa = xa_ref[0]
    sq_a_half = 0.5 * jnp.sum(xa * xa, axis=-1, keepdims=True).T
    dots = jax.lax.dot_general(
        xr, xa, (((1,), (1,)), ((), ())),
        preferred_element_type=jnp.float32)
    s = sq_a_half - dots

    gmin = s[:, :W]
    for g in range(1, G):
        gmin = jnp.minimum(gmin, s[:, g * W:(g + 1) * W])

    big = jnp.float32(jnp.inf)
    t = jnp.full((R, 1), -jnp.inf, dtype=jnp.float32)
    for _ in range(K):
        m = jnp.where(gmin > t, gmin, big)
        t = jnp.min(m, axis=-1, keepdims=True)

    cnt = jnp.sum((s <= t).astype(jnp.float32), axis=-1, keepdims=True)
    over = cnt - jnp.float32(K)

    def cond(carry):
        _, over_c = carry
        return jnp.max(over_c) > 0.0

    def body(carry):
        t_c, over_c = carry
        tm = jnp.max(jnp.where(s < t_c, s, NEG), axis=-1, keepdims=True)
        need = over_c > 0.0
        return (jnp.where(need, tm, t_c),
                jnp.where(need, over_c - 1.0, over_c))

    t, _ = jax.lax.while_loop(cond, body, (t, over))

    out_ref[0] = (s <= t).astype(jnp.float32)


@jax.jit
def kernel(x):
    grid = (B, N // R)
    return pl.pallas_call(
        _adj_kernel,
        grid=grid,
        in_specs=[
            pl.BlockSpec((1, R, C), lambda b, i: (b, i, 0)),
            pl.BlockSpec((1, N, C), lambda b, i: (b, 0, 0)),
        ],
        out_specs=pl.BlockSpec((1, R, N), lambda b, i: (b, i, 0)),
        out_shape=jax.ShapeDtypeStruct((B, N, N), jnp.float32),
        compiler_params=pltpu.CompilerParams(
            dimension_semantics=("parallel", "arbitrary"),
        ),
    )(x, x)
